# baseline (speedup 1.0000x reference)
import functools

import jax
import jax.numpy as jnp
from jax import lax
from jax.experimental import pallas as pl
from jax.experimental.pallas import tpu as pltpu
from jax.experimental.pallas import tpu_sc as plsc

_BLK = 128


def _rank_block(scol_ref, srow_ref):
    a = scol_ref[...]
    b = srow_ref[...]
    n = srow_ref.shape[1]
    j_idx = lax.broadcasted_iota(jnp.int32, (_BLK, n), 1)
    i_idx = pl.program_id(0) * _BLK + lax.broadcasted_iota(jnp.int32, (_BLK, n), 0)
    gt = b > a
    tie = (b == a) & (j_idx < i_idx)
    return jnp.sum((gt | tie).astype(jnp.int32), axis=1, keepdims=True)


def _rank_scale_body(scol_ref, srow_ref, h_ref, rank_ref, hsc_ref):
    rank_ref[...] = _rank_block(scol_ref, srow_ref)
    hsc_ref[...] = h_ref[...] * scol_ref[...]


def _rank_scale(s, h):
    n, d = h.shape
    row_spec = pl.BlockSpec((_BLK, d), lambda i: (i, 0))
    col_spec = pl.BlockSpec((_BLK, 1), lambda i: (i, 0))
    full_spec = pl.BlockSpec((1, n), lambda i: (0, 0))
    return pl.pallas_call(
        _rank_scale_body,
        grid=(n // _BLK,),
        in_specs=[col_spec, full_spec, row_spec],
        out_specs=[col_spec, row_spec],
        out_shape=[
            jax.ShapeDtypeStruct((n, 1), jnp.int32),
            jax.ShapeDtypeStruct((n, d), jnp.float32),
        ],
    )(s.reshape(n, 1), s.reshape(1, n), h)


def _sc_scatter(hsc, rank0, k0):
    n, d = hsc.shape
    info = plsc.get_sparse_core_info()
    nc, ns, lanes = info.num_cores, info.num_subcores, info.num_lanes
    nw = nc * ns
    rpw = n // nw
    mesh = plsc.VectorSubcoreMesh(core_axis_name="c", subcore_axis_name="s")

    @functools.partial(
        pl.kernel,
        mesh=mesh,
        out_type=jax.ShapeDtypeStruct((n, d), jnp.float32),
        scratch_types=[
            pltpu.VMEM((rpw,), jnp.int32),
            pltpu.VMEM((rpw, d), jnp.float32),
            pltpu.SemaphoreType.DMA,
        ],
    )
    def scatter_kernel(h_hbm, r_hbm, out_hbm, idx_v, rows_v, sem):
        wid = lax.axis_index("s") * nc + lax.axis_index("c")
        base = wid * rpw
        pltpu.sync_copy(r_hbm.at[pl.ds(base, rpw)], idx_v)
        pltpu.sync_copy(h_hbm.at[pl.ds(base, rpw)], rows_v)

        trash = k0 + wid

        def clamp_body(c, carry):
            v = idx_v[pl.ds(c * lanes, lanes)]
            idx_v[pl.ds(c * lanes, lanes)] = jnp.where(v < k0, v, trash)
            return carry

        lax.fori_loop(0, rpw // lanes, clamp_body, 0, unroll=True)
        pltpu.async_copy(rows_v, out_hbm.at[idx_v], sem).wait()

    return scatter_kernel(hsc, rank0.reshape(n))


def _rank_combine_body(k0, k1, scol_ref, srow_ref, r0_ref, hsc_ref, hp_ref, out_ref):
    rank1 = _rank_block(scol_ref, srow_ref)
    m1 = rank1 < k1
    m0 = r0_ref[...] < k0
    t0 = jnp.where(m0, hsc_ref[...], 0.0)
    t1 = jnp.where(m1, hp_ref[...] * scol_ref[...], 0.0)
    out_ref[...] = t0 + t1


def _rank_combine(s1p, rank0, hsc, hp0, k0, k1):
    n, d = hsc.shape
    row_spec = pl.BlockSpec((_BLK, d), lambda i: (i, 0))
    col_spec = pl.BlockSpec((_BLK, 1), lambda i: (i, 0))
    full_spec = pl.BlockSpec((1, n), lambda i: (0, 0))
    return pl.pallas_call(
        functools.partial(_rank_combine_body, k0, k1),
        grid=(n // _BLK,),
        in_specs=[col_spec, full_spec, col_spec, row_spec, row_spec],
        out_specs=row_spec,
        out_shape=jax.ShapeDtypeStruct((n, d), jnp.float32),
    )(s1p.reshape(n, 1), s1p.reshape(1, n), rank0, hsc, hp0)


def kernel(g, h, proj_w0, proj_b0, proj_w1, proj_b1):
    n, d = h.shape
    k0 = max(2, int(0.8 * n))
    k1 = max(2, int(0.6 * k0))
    s0 = jax.nn.sigmoid(h @ proj_w0 + proj_b0[0])
    rank0, hsc = _rank_scale(s0, h)
    hp0 = _sc_scatter(hsc, rank0, k0)
    s1 = jax.nn.sigmoid(hp0[:k0] @ proj_w1 + proj_b1[0])
    s1p = jnp.concatenate([s1, jnp.full((n - k0,), -1.0, jnp.float32)])
    return _rank_combine(s1p, rank0, hsc, hp0, k0, k1)


# speedup vs baseline: 1.2700x; 1.2700x over previous
import functools

import jax
import jax.numpy as jnp
from jax import lax
from jax.experimental import pallas as pl
from jax.experimental.pallas import tpu as pltpu
from jax.experimental.pallas import tpu_sc as plsc

_BLK = 128


def _rollm(x, d, axis):
    if axis == 1:
        return jnp.concatenate([x[:, d:], x[:, :d]], axis=1)
    return jnp.concatenate([x[d:, :], x[:d, :]], axis=0)


def _rollp(x, d, axis):
    n = x.shape[axis]
    if axis == 1:
        return jnp.concatenate([x[:, n - d:], x[:, :n - d]], axis=1)
    return jnp.concatenate([x[n - d:, :], x[:n - d, :]], axis=0)


def _bitonic_desc(key):
    r_dim, c_dim = key.shape
    n = r_dim * c_dim
    riota = lax.broadcasted_iota(jnp.int32, (r_dim, c_dim), 0)
    ciota = lax.broadcasted_iota(jnp.int32, (r_dim, c_dim), 1)
    pay = riota * c_dim + ciota
    k = 2
    while k <= n:
        d = k // 2
        while d >= 1:
            if d < c_dim:
                bd = (ciota & d) != 0
                pk = jnp.where(bd, _rollp(key, d, 1), _rollm(key, d, 1))
                pp = jnp.where(bd, _rollp(pay, d, 1), _rollm(pay, d, 1))
            else:
                dr = d // c_dim
                bd = (riota & dr) != 0
                pk = jnp.where(bd, _rollp(key, dr, 0), _rollm(key, dr, 0))
                pp = jnp.where(bd, _rollp(pay, dr, 0), _rollm(pay, dr, 0))
            if k < c_dim:
                bk = (ciota & k) != 0
            else:
                bk = (riota & (k // c_dim)) != 0
            take_min = bk ^ bd
            self_gt = (key > pk) | ((key == pk) & (pay < pp))
            use_partner = self_gt == take_min
            key = jnp.where(use_partner, pk, key)
            pay = jnp.where(use_partner, pp, pay)
            d //= 2
        k *= 2
    return key, pay


def _sort_body(bits_ref, skey_ref, spay_ref):
    skey, spay = _bitonic_desc(bits_ref[...])
    skey_ref[...] = skey
    spay_ref[...] = spay


def _sort_desc(s):
    n = s.shape[0]
    bits = lax.bitcast_convert_type(s, jnp.int32).reshape(n // _BLK, _BLK)
    skey, spay = pl.pallas_call(
        _sort_body,
        out_shape=[
            jax.ShapeDtypeStruct((n // _BLK, _BLK), jnp.int32),
            jax.ShapeDtypeStruct((n // _BLK, _BLK), jnp.int32),
        ],
    )(bits)
    return skey.reshape(n), spay.reshape(n)


def _sc_gather(h, payload):
    n, d = h.shape
    info = plsc.get_sparse_core_info()
    nc, ns = info.num_cores, info.num_subcores
    nw = nc * ns
    rpw = n // nw
    mesh = plsc.VectorSubcoreMesh(core_axis_name="c", subcore_axis_name="s")

    @functools.partial(
        pl.kernel,
        mesh=mesh,
        out_type=jax.ShapeDtypeStruct((n, d), jnp.float32),
        scratch_types=[
            pltpu.VMEM((rpw,), jnp.int32),
            pltpu.VMEM((rpw, d), jnp.float32),
            pltpu.SemaphoreType.DMA,
        ],
    )
    def gather_kernel(h_hbm, pay_hbm, out_hbm, idx_v, rows_v, sem):
        wid = lax.axis_index("s") * nc + lax.axis_index("c")
        base = wid * rpw
        pltpu.sync_copy(pay_hbm.at[pl.ds(base, rpw)], idx_v)
        pltpu.async_copy(h_hbm.at[idx_v], rows_v, sem).wait()
        pltpu.sync_copy(rows_v, out_hbm.at[pl.ds(base, rpw)])

    return gather_kernel(h, payload)


def _combine_body(k0, k1, prm_ref, s0c_ref, h_ref, hg_ref, v0c_ref, s1c_ref, out_ref):
    i_col = pl.program_id(0) * _BLK + lax.broadcasted_iota(jnp.int32, (_BLK, 1), 0)
    b0 = lax.bitcast_convert_type(s0c_ref[...], jnp.int32)
    m0 = (b0 > prm_ref[0]) | ((b0 == prm_ref[0]) & (i_col <= prm_ref[1]))
    b1 = lax.bitcast_convert_type(s1c_ref[...], jnp.int32)
    m1 = (b1 > prm_ref[2]) | ((b1 == prm_ref[2]) & (i_col <= prm_ref[3]))
    t0 = jnp.where(m0, h_ref[...] * s0c_ref[...], 0.0)
    t1 = jnp.where(m1, (hg_ref[...] * v0c_ref[...]) * s1c_ref[...], 0.0)
    out_ref[...] = t0 + t1


def _combine(prm, s0, h, hg, v0, s1p, k0, k1):
    n, d = h.shape
    col = lambda x: x.reshape(n, 1)
    row_spec = pl.BlockSpec((_BLK, d), lambda i: (i, 0))
    col_spec = pl.BlockSpec((_BLK, 1), lambda i: (i, 0))
    return pl.pallas_call(
        functools.partial(_combine_body, k0, k1),
        grid=(n // _BLK,),
        in_specs=[
            pl.BlockSpec(memory_space=pltpu.SMEM),
            col_spec, row_spec, row_spec, col_spec, col_spec,
        ],
        out_specs=row_spec,
        out_shape=jax.ShapeDtypeStruct((n, d), jnp.float32),
    )(prm, col(s0), h, hg, col(v0), col(s1p))


def kernel(g, h, proj_w0, proj_b0, proj_w1, proj_b1):
    n, d = h.shape
    k0 = max(2, int(0.8 * n))
    k1 = max(2, int(0.6 * k0))
    s0 = jax.nn.sigmoid(h @ proj_w0 + proj_b0[0])
    skey0, spay0 = _sort_desc(s0)
    hg = _sc_gather(h, spay0)
    v0 = lax.bitcast_convert_type(skey0, jnp.float32)
    hp0 = hg[:k0] * v0[:k0, None]
    s1 = jax.nn.sigmoid(hp0 @ proj_w1 + proj_b1[0])
    s1p = jnp.concatenate([s1, jnp.zeros((n - k0,), jnp.float32)])
    skey1, spay1 = _sort_desc(s1p)
    prm = jnp.stack([skey0[k0 - 1], spay0[k0 - 1], skey1[k1 - 1], spay1[k1 - 1]])
    return _combine(prm, s0, h, hg, v0, s1p, k0, k1)
